# baseline (speedup 1.0000x reference)
import functools

import jax
import jax.numpy as jnp
from jax import lax
from jax.experimental import pallas as pl
from jax.experimental.pallas import tpu as pltpu
from jax.experimental.pallas import tpu_sc as plsc

B = 16384
EMBED = 64
NC = 2
NS = 16
NW = NC * NS
BPW = B // NW
CH = 128
NCHUNK = BPW // CH

GROUP = 16
NGROUP = BPW // GROUP


@functools.cache
def _make_sc_gather():
    mesh = plsc.VectorSubcoreMesh(core_axis_name="c", subcore_axis_name="s")

    @functools.partial(
        pl.kernel,
        out_type=(
            jax.ShapeDtypeStruct((B, EMBED), jnp.float32),
            jax.ShapeDtypeStruct((B, EMBED), jnp.float32),
        ),
        mesh=mesh,
        scratch_types=[
            pltpu.VMEM((BPW,), jnp.int32),
            pltpu.VMEM((BPW,), jnp.int32),
            pltpu.VMEM((BPW // 2, EMBED), jnp.float32),
            pltpu.VMEM((BPW // 2, EMBED), jnp.float32),
            pltpu.SemaphoreType.DMA,
            pltpu.SemaphoreType.DMA,
        ],
        compiler_params=pltpu.CompilerParams(needs_layout_passes=False),
    )
    def _sc_gather(uid_hbm, iid_hbm, utab_hbm, itab_hbm, uout_hbm, iout_hbm,
                   uidx_v, iidx_v, urows_v, irows_v, sem_s, sem_l):
        wid = lax.axis_index("s") * NC + lax.axis_index("c")
        base = wid * BPW
        pltpu.sync_copy(uid_hbm.at[pl.ds(base, BPW)], uidx_v)
        pltpu.sync_copy(iid_hbm.at[pl.ds(base, BPW)], iidx_v)
        lane = lax.iota(jnp.int32, GROUP)

        S = BPW // 2
        REST = BPW - S

        def fire_local(g, carry):
            gb = S + g * GROUP
            uchunk = uidx_v[pl.ds(gb, GROUP)]
            ichunk = iidx_v[pl.ds(gb, GROUP)]
            for l in range(GROUP):
                ui = jnp.max(jnp.where(lane == l, uchunk, 0))
                ii = jnp.max(jnp.where(lane == l, ichunk, 0))
                b = gb + l
                pltpu.async_copy(
                    utab_hbm.at[pl.ds(ui, 1)],
                    uout_hbm.at[pl.ds(base + b, 1)], sem_l)
                pltpu.async_copy(
                    itab_hbm.at[pl.ds(ii, 1)],
                    iout_hbm.at[pl.ds(base + b, 1)], sem_l)
            return carry

        def fire_stream(g, carry):
            gb = g * GROUP
            uchunk = uidx_v[pl.ds(gb, GROUP)]
            ichunk = iidx_v[pl.ds(gb, GROUP)]
            for l in range(GROUP):
                ui = jnp.max(jnp.where(lane == l, uchunk, 0))
                ii = jnp.max(jnp.where(lane == l, ichunk, 0))
                r = gb + l
                pltpu.async_copy(
                    utab_hbm.at[pl.ds(ui, 1)],
                    urows_v.at[pl.ds(r, 1)], sem_s)
                pltpu.async_copy(
                    itab_hbm.at[pl.ds(ii, 1)],
                    irows_v.at[pl.ds(r, 1)], sem_s)
            return carry

        lax.fori_loop(0, REST // GROUP, fire_local, 0)
        lax.fori_loop(0, S // GROUP, fire_stream, 0)
        pltpu.make_async_copy(uout_hbm.at[pl.ds(base, S)],
                              urows_v, sem_s).wait()
        pltpu.make_async_copy(iout_hbm.at[pl.ds(base, S)],
                              irows_v, sem_s).wait()
        pltpu.sync_copy(urows_v, uout_hbm.at[pl.ds(base, S)])
        pltpu.sync_copy(irows_v, iout_hbm.at[pl.ds(base, S)])
        pltpu.make_async_copy(uout_hbm.at[pl.ds(base + S, REST)],
                              uout_hbm.at[pl.ds(base + S, REST)], sem_l).wait()
        pltpu.make_async_copy(iout_hbm.at[pl.ds(base + S, REST)],
                              iout_hbm.at[pl.ds(base + S, REST)], sem_l).wait()

    return _sc_gather


def _mlp_body(ue_ref, ie_ref, feat_ref, w1u_ref, w1i_ref, w1f_ref,
              b1_ref, gamma_ref, beta_ref, w2_ref, b2_ref, out_ref):
    h = (jnp.dot(ue_ref[...], w1u_ref[...], preferred_element_type=jnp.float32)
         + jnp.dot(ie_ref[...], w1i_ref[...], preferred_element_type=jnp.float32)
         + jnp.dot(feat_ref[...], w1f_ref[...], preferred_element_type=jnp.float32)
         + b1_ref[...])
    mean = jnp.mean(h, axis=0, keepdims=True)
    d = h - mean
    var = jnp.mean(d * d, axis=0, keepdims=True)
    hn = d * lax.rsqrt(var + 1e-5) * gamma_ref[...] + beta_ref[...]
    hn = jnp.maximum(hn, 0.0)
    out_ref[...] = (jnp.sum(hn * w2_ref[...], axis=1, keepdims=True)
                    + b2_ref[...])


_mlp = pl.pallas_call(
    _mlp_body,
    out_shape=jax.ShapeDtypeStruct((B, 1), jnp.float32),
)


def kernel(user_id, item_id, users_info, items_info, user_table, item_table,
           W1, b1, gamma, beta, W2, b2):
    ue, ie = _make_sc_gather()(user_id, item_id, user_table, item_table)
    feats = jnp.concatenate([users_info, items_info], axis=1)
    return _mlp(ue, ie, feats,
                W1[:EMBED], W1[EMBED:2 * EMBED], W1[2 * EMBED:],
                b1.reshape(1, -1), gamma.reshape(1, -1), beta.reshape(1, -1),
                W2.reshape(1, -1), b2.reshape(1, 1))


# speedup vs baseline: 1.1519x; 1.1519x over previous
import functools

import jax
import jax.numpy as jnp
from jax import lax
from jax.experimental import pallas as pl
from jax.experimental.pallas import tpu as pltpu
from jax.experimental.pallas import tpu_sc as plsc

B = 16384
EMBED = 64
NC = 2
NS = 16
NW = NC * NS
BPW = B // NW
CH = 128
NCHUNK = BPW // CH

GROUP = 16
NGROUP = BPW // GROUP


@functools.cache
def _make_sc_gather():
    mesh = plsc.VectorSubcoreMesh(core_axis_name="c", subcore_axis_name="s")

    @functools.partial(
        pl.kernel,
        out_type=(
            jax.ShapeDtypeStruct((B, EMBED), jnp.float32),
            jax.ShapeDtypeStruct((B, EMBED), jnp.float32),
        ),
        mesh=mesh,
        scratch_types=[
            pltpu.VMEM((BPW,), jnp.int32),
            pltpu.VMEM((BPW,), jnp.int32),
            pltpu.VMEM((384, EMBED), jnp.float32),
            pltpu.VMEM((384, EMBED), jnp.float32),
            pltpu.SemaphoreType.DMA,
            pltpu.SemaphoreType.DMA,
        ],
        compiler_params=pltpu.CompilerParams(needs_layout_passes=False),
    )
    def _sc_gather(uid_hbm, iid_hbm, utab_hbm, itab_hbm, uout_hbm, iout_hbm,
                   uidx_v, iidx_v, urows_v, irows_v, sem_s, sem_l):
        wid = lax.axis_index("s") * NC + lax.axis_index("c")
        base = wid * BPW
        pltpu.sync_copy(uid_hbm.at[pl.ds(base, BPW)], uidx_v)
        pltpu.sync_copy(iid_hbm.at[pl.ds(base, BPW)], iidx_v)
        lane = lax.iota(jnp.int32, GROUP)

        S = 384
        REST = BPW - S

        def fire_local(g, carry):
            gb = S + g * GROUP
            uchunk = uidx_v[pl.ds(gb, GROUP)]
            ichunk = iidx_v[pl.ds(gb, GROUP)]
            for l in range(GROUP):
                ui = jnp.max(jnp.where(lane == l, uchunk, 0))
                ii = jnp.max(jnp.where(lane == l, ichunk, 0))
                b = gb + l
                pltpu.async_copy(
                    utab_hbm.at[pl.ds(ui, 1)],
                    uout_hbm.at[pl.ds(base + b, 1)], sem_l)
                pltpu.async_copy(
                    itab_hbm.at[pl.ds(ii, 1)],
                    iout_hbm.at[pl.ds(base + b, 1)], sem_l)
            return carry

        def fire_stream(g, carry):
            gb = g * GROUP
            uchunk = uidx_v[pl.ds(gb, GROUP)]
            ichunk = iidx_v[pl.ds(gb, GROUP)]
            for l in range(GROUP):
                ui = jnp.max(jnp.where(lane == l, uchunk, 0))
                ii = jnp.max(jnp.where(lane == l, ichunk, 0))
                r = gb + l
                pltpu.async_copy(
                    utab_hbm.at[pl.ds(ui, 1)],
                    urows_v.at[pl.ds(r, 1)], sem_s)
                pltpu.async_copy(
                    itab_hbm.at[pl.ds(ii, 1)],
                    irows_v.at[pl.ds(r, 1)], sem_s)
            return carry

        lax.fori_loop(0, REST // GROUP, fire_local, 0)
        lax.fori_loop(0, S // GROUP, fire_stream, 0)
        pltpu.make_async_copy(uout_hbm.at[pl.ds(base, S)],
                              urows_v, sem_s).wait()
        pltpu.make_async_copy(iout_hbm.at[pl.ds(base, S)],
                              irows_v, sem_s).wait()
        pltpu.sync_copy(urows_v, uout_hbm.at[pl.ds(base, S)])
        pltpu.sync_copy(irows_v, iout_hbm.at[pl.ds(base, S)])
        pltpu.make_async_copy(uout_hbm.at[pl.ds(base + S, REST)],
                              uout_hbm.at[pl.ds(base + S, REST)], sem_l).wait()
        pltpu.make_async_copy(iout_hbm.at[pl.ds(base + S, REST)],
                              iout_hbm.at[pl.ds(base + S, REST)], sem_l).wait()

    return _sc_gather


def _mlp_body(ue_ref, ie_ref, feat_ref, w1u_ref, w1i_ref, w1f_ref,
              b1_ref, gamma_ref, beta_ref, w2_ref, b2_ref, out_ref):
    h = (jnp.dot(ue_ref[...], w1u_ref[...], preferred_element_type=jnp.float32)
         + jnp.dot(ie_ref[...], w1i_ref[...], preferred_element_type=jnp.float32)
         + jnp.dot(feat_ref[...], w1f_ref[...], preferred_element_type=jnp.float32)
         + b1_ref[...])
    mean = jnp.mean(h, axis=0, keepdims=True)
    d = h - mean
    var = jnp.mean(d * d, axis=0, keepdims=True)
    hn = d * lax.rsqrt(var + 1e-5) * gamma_ref[...] + beta_ref[...]
    hn = jnp.maximum(hn, 0.0)
    out_ref[...] = (jnp.sum(hn * w2_ref[...], axis=1, keepdims=True)
                    + b2_ref[...])


_mlp = pl.pallas_call(
    _mlp_body,
    out_shape=jax.ShapeDtypeStruct((B, 1), jnp.float32),
)


def kernel(user_id, item_id, users_info, items_info, user_table, item_table,
           W1, b1, gamma, beta, W2, b2):
    ue, ie = _make_sc_gather()(user_id, item_id, user_table, item_table)
    feats = jnp.concatenate([users_info, items_info], axis=1)
    return _mlp(ue, ie, feats,
                W1[:EMBED], W1[EMBED:2 * EMBED], W1[2 * EMBED:],
                b1.reshape(1, -1), gamma.reshape(1, -1), beta.reshape(1, -1),
                W2.reshape(1, -1), b2.reshape(1, 1))


# speedup vs baseline: 1.2229x; 1.0617x over previous
import functools

import jax
import jax.numpy as jnp
from jax import lax
from jax.experimental import pallas as pl
from jax.experimental.pallas import tpu as pltpu
from jax.experimental.pallas import tpu_sc as plsc

B = 16384
EMBED = 64
NC = 2
NS = 16
NW = NC * NS
KTC = B // 2
KSC = B - KTC
BPW = KSC // NW
GROUP = 16


@functools.cache
def _make_sc_gather():
    mesh = plsc.VectorSubcoreMesh(core_axis_name="c", subcore_axis_name="s")

    @functools.partial(
        pl.kernel,
        out_type=(
            jax.ShapeDtypeStruct((KSC, EMBED), jnp.float32),
            jax.ShapeDtypeStruct((KSC, EMBED), jnp.float32),
        ),
        mesh=mesh,
        scratch_types=[
            pltpu.VMEM((BPW,), jnp.int32),
            pltpu.VMEM((BPW,), jnp.int32),
            pltpu.VMEM((BPW, EMBED), jnp.float32),
            pltpu.VMEM((BPW, EMBED), jnp.float32),
            pltpu.SemaphoreType.DMA,
        ],
        compiler_params=pltpu.CompilerParams(needs_layout_passes=False),
    )
    def _sc_gather(uid_hbm, iid_hbm, utab_hbm, itab_hbm, uout_hbm, iout_hbm,
                   uidx_v, iidx_v, urows_v, irows_v, sem):
        wid = lax.axis_index("s") * NC + lax.axis_index("c")
        base = wid * BPW
        pltpu.sync_copy(uid_hbm.at[pl.ds(base, BPW)], uidx_v)
        pltpu.sync_copy(iid_hbm.at[pl.ds(base, BPW)], iidx_v)
        lane = lax.iota(jnp.int32, GROUP)

        def body(g, carry):
            gb = g * GROUP
            uchunk = uidx_v[pl.ds(gb, GROUP)]
            ichunk = iidx_v[pl.ds(gb, GROUP)]
            for l in range(GROUP):
                ui = jnp.max(jnp.where(lane == l, uchunk, 0))
                ii = jnp.max(jnp.where(lane == l, ichunk, 0))
                r = gb + l
                pltpu.async_copy(
                    utab_hbm.at[pl.ds(ui, 1)],
                    urows_v.at[pl.ds(r, 1)], sem)
                pltpu.async_copy(
                    itab_hbm.at[pl.ds(ii, 1)],
                    irows_v.at[pl.ds(r, 1)], sem)
            return carry

        lax.fori_loop(0, BPW // GROUP, body, 0)
        pltpu.make_async_copy(uout_hbm.at[pl.ds(base, BPW)],
                              urows_v, sem).wait()
        pltpu.make_async_copy(iout_hbm.at[pl.ds(base, BPW)],
                              irows_v, sem).wait()
        pltpu.sync_copy(urows_v, uout_hbm.at[pl.ds(base, BPW)])
        pltpu.sync_copy(irows_v, iout_hbm.at[pl.ds(base, BPW)])

    return _sc_gather


def _tc_gather_body(uid_ref, iid_ref, utab_ref, itab_ref, ue_ref, ie_ref,
                    sem):
    def body(b, carry):
        pltpu.make_async_copy(
            utab_ref.at[pl.ds(uid_ref[b], 1)],
            ue_ref.at[pl.ds(b, 1)], sem).start()
        pltpu.make_async_copy(
            itab_ref.at[pl.ds(iid_ref[b], 1)],
            ie_ref.at[pl.ds(b, 1)], sem).start()
        return carry

    lax.fori_loop(0, KTC, body, 0)
    pltpu.make_async_copy(utab_ref.at[pl.ds(0, KTC)], ue_ref, sem).wait()
    pltpu.make_async_copy(itab_ref.at[pl.ds(0, KTC)], ie_ref, sem).wait()


_tc_gather = pl.pallas_call(
    _tc_gather_body,
    in_specs=[
        pl.BlockSpec(memory_space=pltpu.SMEM),
        pl.BlockSpec(memory_space=pltpu.SMEM),
        pl.BlockSpec(memory_space=pltpu.MemorySpace.HBM),
        pl.BlockSpec(memory_space=pltpu.MemorySpace.HBM),
    ],
    out_shape=(
        jax.ShapeDtypeStruct((KTC, EMBED), jnp.float32),
        jax.ShapeDtypeStruct((KTC, EMBED), jnp.float32),
    ),
    scratch_shapes=[pltpu.SemaphoreType.DMA],
)


def _mlp_body(uet_ref, iet_ref, ues_ref, ies_ref, feat_ref,
              w1u_ref, w1i_ref, w1f_ref,
              b1_ref, gamma_ref, beta_ref, w2_ref, b2_ref, out_ref):
    h_top = (jnp.dot(uet_ref[...], w1u_ref[...],
                     preferred_element_type=jnp.float32)
             + jnp.dot(iet_ref[...], w1i_ref[...],
                       preferred_element_type=jnp.float32))
    h_bot = (jnp.dot(ues_ref[...], w1u_ref[...],
                     preferred_element_type=jnp.float32)
             + jnp.dot(ies_ref[...], w1i_ref[...],
                       preferred_element_type=jnp.float32))
    h = (jnp.concatenate([h_top, h_bot], axis=0)
         + jnp.dot(feat_ref[...], w1f_ref[...],
                   preferred_element_type=jnp.float32)
         + b1_ref[...])
    mean = jnp.mean(h, axis=0, keepdims=True)
    d = h - mean
    var = jnp.mean(d * d, axis=0, keepdims=True)
    hn = d * lax.rsqrt(var + 1e-5) * gamma_ref[...] + beta_ref[...]
    hn = jnp.maximum(hn, 0.0)
    out_ref[...] = (jnp.sum(hn * w2_ref[...], axis=1, keepdims=True)
                    + b2_ref[...])


_mlp = pl.pallas_call(
    _mlp_body,
    out_shape=jax.ShapeDtypeStruct((B, 1), jnp.float32),
)


def kernel(user_id, item_id, users_info, items_info, user_table, item_table,
           W1, b1, gamma, beta, W2, b2):
    ue_sc, ie_sc = _make_sc_gather()(user_id[KTC:], item_id[KTC:],
                                     user_table, item_table)
    ue_tc, ie_tc = _tc_gather(user_id[:KTC], item_id[:KTC],
                              user_table, item_table)
    feats = jnp.concatenate([users_info, items_info], axis=1)
    return _mlp(ue_tc, ie_tc, ue_sc, ie_sc, feats,
                W1[:EMBED], W1[EMBED:2 * EMBED], W1[2 * EMBED:],
                b1.reshape(1, -1), gamma.reshape(1, -1), beta.reshape(1, -1),
                W2.reshape(1, -1), b2.reshape(1, 1))


# speedup vs baseline: 1.3267x; 1.0849x over previous
import functools

import jax
import jax.numpy as jnp
from jax import lax
from jax.experimental import pallas as pl
from jax.experimental.pallas import tpu as pltpu
from jax.experimental.pallas import tpu_sc as plsc

B = 16384
EMBED = 64
NC = 2
NS = 16
NW = NC * NS
BPW = B // NW
GROUP = 16


@functools.cache
def _make_sc_gather():
    mesh = plsc.VectorSubcoreMesh(core_axis_name="c", subcore_axis_name="s")

    @functools.partial(
        pl.kernel,
        out_type=(
            jax.ShapeDtypeStruct((B, EMBED), jnp.float32),
            jax.ShapeDtypeStruct((B, EMBED), jnp.float32),
        ),
        mesh=mesh,
        scratch_types=[
            pltpu.VMEM((BPW,), jnp.int32),
            pltpu.VMEM((BPW,), jnp.int32),
            pltpu.VMEM((BPW // 2, EMBED), jnp.float32),
            pltpu.VMEM((BPW // 2, EMBED), jnp.float32),
            pltpu.SemaphoreType.DMA,
        ],
        compiler_params=pltpu.CompilerParams(needs_layout_passes=False),
    )
    def _sc_gather(uid_hbm, iid_hbm, utab_hbm, itab_hbm, uout_hbm, iout_hbm,
                   uidx_v, iidx_v, urows_v, irows_v, sem):
        wid = lax.axis_index("s") * NC + lax.axis_index("c")
        base = wid * BPW
        pltpu.sync_copy(uid_hbm.at[pl.ds(base, BPW)], uidx_v)
        pltpu.sync_copy(iid_hbm.at[pl.ds(base, BPW)], iidx_v)
        lane = lax.iota(jnp.int32, GROUP)

        HALF = BPW // 2

        for half in range(2):
            hbase = half * HALF

            def body(g, carry):
                gb = hbase + g * GROUP
                uchunk = uidx_v[pl.ds(gb, GROUP)]
                ichunk = iidx_v[pl.ds(gb, GROUP)]
                for l in range(GROUP):
                    ui = jnp.max(jnp.where(lane == l, uchunk, 0))
                    ii = jnp.max(jnp.where(lane == l, ichunk, 0))
                    r = g * GROUP + l
                    pltpu.async_copy(
                        utab_hbm.at[pl.ds(ui, 1)],
                        urows_v.at[pl.ds(r, 1)], sem)
                    pltpu.async_copy(
                        itab_hbm.at[pl.ds(ii, 1)],
                        irows_v.at[pl.ds(r, 1)], sem)
                return carry

            lax.fori_loop(0, HALF // GROUP, body, 0)
            pltpu.make_async_copy(uout_hbm.at[pl.ds(base, HALF)],
                                  urows_v, sem).wait()
            pltpu.make_async_copy(iout_hbm.at[pl.ds(base, HALF)],
                                  irows_v, sem).wait()
            pltpu.sync_copy(urows_v, uout_hbm.at[pl.ds(base + hbase, HALF)])
            pltpu.sync_copy(irows_v, iout_hbm.at[pl.ds(base + hbase, HALF)])

    return _sc_gather


def _mlp_body(ue_ref, ie_ref, feat_ref, w1u_ref, w1i_ref, w1f_ref,
              b1_ref, gamma_ref, beta_ref, w2_ref, b2_ref, out_ref):
    h = (jnp.dot(ue_ref[...], w1u_ref[...], preferred_element_type=jnp.float32)
         + jnp.dot(ie_ref[...], w1i_ref[...], preferred_element_type=jnp.float32)
         + jnp.dot(feat_ref[...], w1f_ref[...], preferred_element_type=jnp.float32)
         + b1_ref[...])
    mean = jnp.mean(h, axis=0, keepdims=True)
    d = h - mean
    var = jnp.mean(d * d, axis=0, keepdims=True)
    hn = d * lax.rsqrt(var + 1e-5) * gamma_ref[...] + beta_ref[...]
    hn = jnp.maximum(hn, 0.0)
    out_ref[...] = (jnp.sum(hn * w2_ref[...], axis=1, keepdims=True)
                    + b2_ref[...])


_mlp = pl.pallas_call(
    _mlp_body,
    out_shape=jax.ShapeDtypeStruct((B, 1), jnp.float32),
)


def kernel(user_id, item_id, users_info, items_info, user_table, item_table,
           W1, b1, gamma, beta, W2, b2):
    ue, ie = _make_sc_gather()(user_id, item_id, user_table, item_table)
    feats = jnp.concatenate([users_info, items_info], axis=1)
    return _mlp(ue, ie, feats,
                W1[:EMBED], W1[EMBED:2 * EMBED], W1[2 * EMBED:],
                b1.reshape(1, -1), gamma.reshape(1, -1), beta.reshape(1, -1),
                W2.reshape(1, -1), b2.reshape(1, 1))
